# baseline (speedup 1.0000x reference)
import functools

import numpy as np
import jax
import jax.numpy as jnp
from jax import lax
from jax.experimental import pallas as pl
from jax.experimental.pallas import tpu as pltpu
from jax.experimental.pallas import tpu_sc as plsc

_RING = [k for k in range(64) if not (2 <= k // 8 <= 5 and 2 <= k % 8 <= 5)]
_TOKEN_TILE = ([(0, k) for k in range(64)]
               + [(1, k) for k in _RING]
               + [(2, k) for k in _RING])

_NW = 32
_B = 160 * 3 * 16
_BPW = _B // _NW
_CHUNK = 120
_NCHUNK = _BPW // _CHUNK


def _build_row_idx():
    idx = np.empty((_B,), np.int32)
    for r in range(_B):
        n, c, a = r // 48, (r // 16) % 3, r % 16
        lvl, k = _TOKEN_TILE[n]
        y, x = k // 8, k % 8
        idx[r] = ((lvl * 3 + c) * 128 + 16 * y + a) * 8 + x
    return idx.reshape(_NW, _NCHUNK, _CHUNK)


_ROW_IDX = _build_row_idx()


def _pool_kernel(img_ref, planes_ref):
    r = lax.broadcasted_iota(jnp.int32, (128, 512), 0)
    c = lax.broadcasted_iota(jnp.int32, (128, 512), 1)
    a4 = (c // 4 == r).astype(jnp.float32)
    a2 = (c[:, :256] // 2 == r[:, :256]).astype(jnp.float32)

    def dot(x, y, dims):
        return lax.dot_general(x, y, (dims, ((), ())),
                               precision=lax.Precision.HIGHEST,
                               preferred_element_type=jnp.float32)

    for ch in range(3):
        img = img_ref[ch]
        planes_ref[ch] = jnp.floor(img[192:320, 192:320])
        sub = img[128:384, 128:384]
        t1 = dot(a2, sub, ((1,), (0,)))
        planes_ref[3 + ch] = jnp.floor(dot(t1, a2, ((1,), (1,))) * 0.25)
        t2 = dot(a4, img, ((1,), (0,)))
        planes_ref[6 + ch] = jnp.floor(dot(t2, a4, ((1,), (1,))) * 0.0625)


_TOK_PER_W = 160 // _NW


def _token_geom(n):
    lvl = jnp.where(n < 64, 0, jnp.where(n < 112, 1, 2))
    p = jnp.where(n < 64, n, jnp.where(n < 112, n - 64, n - 112))
    m = (p - 16) % 4
    k_mid = (2 + (p - 16) // 4) * 8 + jnp.where(m < 2, m, m + 4)
    k_ring = jnp.where(p < 16, p, jnp.where(p < 32, k_mid, p + 16))
    k = jnp.where(n < 64, p, k_ring)
    return lvl, k // 8, k % 8


@functools.cache
def _make_gather():
    @functools.partial(
        pl.kernel,
        out_type=jax.ShapeDtypeStruct((160, 3, 16, 16), jnp.float32),
        scratch_types=[
            pltpu.VMEM((_TOK_PER_W, 3, 16, 128), jnp.float32),
            pltpu.VMEM((_TOK_PER_W, 3, 16, 16), jnp.float32),
            pltpu.SemaphoreType.DMA,
            pltpu.SemaphoreType.DMA,
        ],
        mesh=plsc.VectorSubcoreMesh(core_axis_name="c", subcore_axis_name="s"),
    )
    def _gather_kernel(planes_hbm, out_hbm, stage_v, tok_v, gsem, ssem):
        wid = lax.axis_index("s") * 2 + lax.axis_index("c")
        geoms = []
        gathers = []
        for t in range(_TOK_PER_W):
            n = _TOK_PER_W * wid + t
            lvl, y, x = _token_geom(n)
            geoms.append((n, x))
            gathers.append(pltpu.async_copy(
                planes_hbm.at[pl.ds(pl.multiple_of(3 * lvl, 3), 3),
                              pl.ds(pl.multiple_of(16 * y, 16), 16), :],
                stage_v.at[t], gsem))
        for t, cp in enumerate(gathers):
            cp.wait()
            n, x = geoms[t]
            col = pl.multiple_of(16 * x, 16)
            for c in range(3):
                for a in range(16):
                    tok_v[t, c, a, :] = stage_v[t, c, a, pl.ds(col, 16)]
        stores = [
            pltpu.async_copy(tok_v.at[t], out_hbm.at[geoms[t][0]], ssem)
            for t in range(_TOK_PER_W)
        ]
        for cp in stores:
            cp.wait()

    return _gather_kernel


def kernel(images):
    planes = pl.pallas_call(
        _pool_kernel,
        out_shape=jax.ShapeDtypeStruct((9, 128, 128), jnp.float32),
    )(images)
    return _make_gather()(planes)


# speedup vs baseline: 1.1012x; 1.1012x over previous
import functools

import jax
import jax.numpy as jnp
from jax import lax
from jax.experimental import pallas as pl
from jax.experimental.pallas import tpu as pltpu
from jax.experimental.pallas import tpu_sc as plsc

_RING_SLICES = ((0, 18), (22, 26), (30, 34), (38, 42), (46, 64))

_NW = 32


def _pool_kernel(img_ref, out_ref):
    r = lax.broadcasted_iota(jnp.int32, (512, 128), 0)
    c = lax.broadcasted_iota(jnp.int32, (512, 128), 1)
    s4 = (r // 4 == c).astype(jnp.float32)
    s2 = (r[:256] // 2 == c[:256]).astype(jnp.float32)

    def dot(x, y, cdims):
        return lax.dot_general(x, y, ((cdims, ((), ()))),
                               precision=lax.Precision.HIGHEST,
                               preferred_element_type=jnp.float32)

    for ch in range(3):
        img = img_ref[ch]
        sub = img[128:384, 128:384]
        rows2 = dot(s2, sub, ((0,), (0,)))
        p1 = jnp.floor(dot(rows2, s2, ((1,), (0,))) * 0.25)
        rows4 = dot(s4, img, ((0,), (0,)))
        p2 = jnp.floor(dot(rows4, s4, ((1,), (0,))) * 0.0625)

        for base, plane in ((0, p1), (48, p2)):
            tiles = plane.reshape(8, 16, 8, 16).transpose(0, 2, 1, 3)
            tiles = tiles.reshape(64, 16, 16)
            off = base
            for s0, s1 in _RING_SLICES:
                out_ref[off:off + (s1 - s0), ch] = tiles[s0:s1]
                off += s1 - s0


_TOK0_PER_W = 64 // _NW


@functools.cache
def _make_l0_gather():
    @functools.partial(
        pl.kernel,
        out_type=jax.ShapeDtypeStruct((64, 3, 16, 16), jnp.float32),
        scratch_types=[
            pltpu.VMEM((_TOK0_PER_W, 3, 16, 128), jnp.float32),
            pltpu.VMEM((_TOK0_PER_W, 3, 16, 16), jnp.float32),
            pltpu.SemaphoreType.DMA,
            pltpu.SemaphoreType.DMA,
        ],
        mesh=plsc.VectorSubcoreMesh(core_axis_name="c", subcore_axis_name="s"),
    )
    def _l0_gather(img_hbm, out_hbm, stage_v, tok_v, gsem, ssem):
        wid = lax.axis_index("s") * 2 + lax.axis_index("c")
        geoms = []
        gathers = []
        for t in range(_TOK0_PER_W):
            n = _TOK0_PER_W * wid + t
            y, x = n // 8, n % 8
            win = jnp.where(x < 4, 128, 256)
            geoms.append((n, 192 + 16 * x - win))
            gathers.append(pltpu.async_copy(
                img_hbm.at[:, pl.ds(pl.multiple_of(192 + 16 * y, 16), 16),
                           pl.ds(pl.multiple_of(win, 128), 128)],
                stage_v.at[t], gsem))
        stores = []
        for t, cp in enumerate(gathers):
            cp.wait()
            n, off = geoms[t]
            col = pl.multiple_of(off, 16)
            for c in range(3):
                for a in range(16):
                    v = stage_v[t, c, a, pl.ds(col, 16)]
                    tok_v[t, c, a, :] = v.astype(jnp.int32).astype(jnp.float32)
            stores.append(pltpu.async_copy(tok_v.at[t], out_hbm.at[n], ssem))
        for cp in stores:
            cp.wait()

    return _l0_gather


def kernel(images):
    tok0 = _make_l0_gather()(images)
    tok12 = pl.pallas_call(
        _pool_kernel,
        out_shape=jax.ShapeDtypeStruct((96, 3, 16, 16), jnp.float32),
    )(images)
    return jnp.concatenate([tok0, tok12], axis=0)


# speedup vs baseline: 1.1019x; 1.0007x over previous
import functools

import jax
import jax.numpy as jnp
from jax import lax
from jax.experimental import pallas as pl
from jax.experimental.pallas import tpu as pltpu
from jax.experimental.pallas import tpu_sc as plsc

_RING_SLICES = ((0, 18), (22, 26), (30, 34), (38, 42), (46, 64))

_NW = 32


def _pool_kernel(img_ref, out_ref):
    r = lax.broadcasted_iota(jnp.int32, (512, 128), 0)
    c = lax.broadcasted_iota(jnp.int32, (512, 128), 1)
    s4 = (r // 4 == c).astype(jnp.float32)
    s2 = (r[:256] // 2 == c[:256]).astype(jnp.float32)

    def dot(x, y, cdims):
        return lax.dot_general(x, y, ((cdims, ((), ()))),
                               precision=lax.Precision.HIGHEST,
                               preferred_element_type=jnp.float32)

    for ch in range(3):
        img = img_ref[ch]
        sub = img[128:384, 128:384]
        rows2 = dot(s2, sub, ((0,), (0,)))
        p1 = jnp.floor(dot(rows2, s2, ((1,), (0,))) * 0.25)
        rows4 = dot(s4, img, ((0,), (0,)))
        p2 = jnp.floor(dot(rows4, s4, ((1,), (0,))) * 0.0625)

        for base, plane in ((0, p1), (48, p2)):
            tiles = plane.reshape(8, 16, 8, 16).transpose(0, 2, 1, 3)
            tiles = tiles.reshape(64, 16, 16)
            off = base
            for s0, s1 in _RING_SLICES:
                out_ref[off:off + (s1 - s0), ch] = tiles[s0:s1]
                off += s1 - s0


_TOK0_PER_W = 64 // _NW


@functools.cache
def _make_l0_gather():
    @functools.partial(
        pl.kernel,
        out_type=jax.ShapeDtypeStruct((64, 3, 16, 16), jnp.float32),
        scratch_types=[
            pltpu.VMEM((1, 3, 16, 128), jnp.float32),
            pltpu.VMEM((_TOK0_PER_W, 3, 16, 16), jnp.float32),
            pltpu.SemaphoreType.DMA,
            pltpu.SemaphoreType.DMA,
        ],
        mesh=plsc.VectorSubcoreMesh(core_axis_name="c", subcore_axis_name="s"),
    )
    def _l0_gather(img_hbm, out_hbm, stage_v, tok_v, gsem, ssem):
        wid = lax.axis_index("s") * 2 + lax.axis_index("c")
        n0 = 2 * wid
        y, x0 = n0 // 8, n0 % 8
        win = jnp.where(x0 < 4, 128, 256)
        pltpu.async_copy(
            img_hbm.at[:, pl.ds(pl.multiple_of(192 + 16 * y, 16), 16),
                       pl.ds(pl.multiple_of(win, 128), 128)],
            stage_v.at[0], gsem).wait()
        stores = []
        for t in range(_TOK0_PER_W):
            col = pl.multiple_of(192 + 16 * (x0 + t) - win, 16)
            for c in range(3):
                for a in range(16):
                    v = stage_v[0, c, a, pl.ds(col, 16)]
                    tok_v[t, c, a, :] = v.astype(jnp.int32).astype(jnp.float32)
            stores.append(pltpu.async_copy(tok_v.at[t], out_hbm.at[n0 + t], ssem))
        for cp in stores:
            cp.wait()

    return _l0_gather


def kernel(images):
    tok0 = _make_l0_gather()(images)
    tok12 = pl.pallas_call(
        _pool_kernel,
        out_shape=jax.ShapeDtypeStruct((96, 3, 16, 16), jnp.float32),
    )(images)
    return jnp.concatenate([tok0, tok12], axis=0)


# speedup vs baseline: 1.1401x; 1.0346x over previous
import functools

import jax
import jax.numpy as jnp
from jax import lax
from jax.experimental import pallas as pl
from jax.experimental.pallas import tpu as pltpu
from jax.experimental.pallas import tpu_sc as plsc

_RING_SLICES = ((0, 18), (22, 26), (30, 34), (38, 42), (46, 64))

_NW = 32


def _pool_kernel(img_ref, out_ref):
    r = lax.broadcasted_iota(jnp.int32, (512, 128), 0)
    c = lax.broadcasted_iota(jnp.int32, (512, 128), 1)
    s4 = (r // 4 == c).astype(jnp.float32)
    s2 = (r[:256] // 2 == c[:256]).astype(jnp.float32)

    s4b = s4.astype(jnp.bfloat16)
    s2b = s2.astype(jnp.bfloat16)

    def split3(x):
        hi = x.astype(jnp.bfloat16)
        r = x - hi.astype(jnp.float32)
        mid = r.astype(jnp.bfloat16)
        lo = (r - mid.astype(jnp.float32)).astype(jnp.bfloat16)
        return hi, mid, lo

    def poolrows(s, x):
        parts = [lax.dot_general(s, p, ((((0,), (0,))), ((), ())),
                                 preferred_element_type=jnp.float32)
                 for p in split3(x)]
        return (parts[0] + parts[1]) + parts[2]

    def poolcols(x, s):
        parts = [lax.dot_general(p, s, ((((1,), (0,))), ((), ())),
                                 preferred_element_type=jnp.float32)
                 for p in split3(x)]
        return (parts[0] + parts[1]) + parts[2]

    for ch in range(3):
        img = img_ref[ch]
        sub = img[128:384, 128:384]
        rows2 = poolrows(s2b, sub)
        p1 = jnp.floor(poolcols(rows2, s2b) * 0.25)
        rows4 = poolrows(s4b, img)
        p2 = jnp.floor(poolcols(rows4, s4b) * 0.0625)

        for base, plane in ((0, p1), (48, p2)):
            tiles = plane.reshape(8, 16, 8, 16).transpose(0, 2, 1, 3)
            tiles = tiles.reshape(64, 16, 16)
            off = base
            for s0, s1 in _RING_SLICES:
                out_ref[off:off + (s1 - s0), ch] = tiles[s0:s1]
                off += s1 - s0


_TOK0_PER_W = 64 // _NW


@functools.cache
def _make_l0_gather():
    @functools.partial(
        pl.kernel,
        out_type=jax.ShapeDtypeStruct((64, 3, 16, 16), jnp.float32),
        scratch_types=[
            pltpu.VMEM((1, 3, 16, 128), jnp.float32),
            pltpu.VMEM((_TOK0_PER_W, 3, 16, 16), jnp.float32),
            pltpu.SemaphoreType.DMA,
            pltpu.SemaphoreType.DMA,
        ],
        mesh=plsc.VectorSubcoreMesh(core_axis_name="c", subcore_axis_name="s"),
    )
    def _l0_gather(img_hbm, out_hbm, stage_v, tok_v, gsem, ssem):
        wid = lax.axis_index("s") * 2 + lax.axis_index("c")
        n0 = 2 * wid
        y, x0 = n0 // 8, n0 % 8
        win = jnp.where(x0 < 4, 128, 256)
        pltpu.async_copy(
            img_hbm.at[:, pl.ds(pl.multiple_of(192 + 16 * y, 16), 16),
                       pl.ds(pl.multiple_of(win, 128), 128)],
            stage_v.at[0], gsem).wait()
        stores = []
        for t in range(_TOK0_PER_W):
            col = pl.multiple_of(192 + 16 * (x0 + t) - win, 16)
            for c in range(3):
                for a in range(16):
                    v = stage_v[0, c, a, pl.ds(col, 16)]
                    tok_v[t, c, a, :] = v.astype(jnp.int32).astype(jnp.float32)
            stores.append(pltpu.async_copy(tok_v.at[t], out_hbm.at[n0 + t], ssem))
        for cp in stores:
            cp.wait()

    return _l0_gather


def kernel(images):
    tok0 = _make_l0_gather()(images)
    tok12 = pl.pallas_call(
        _pool_kernel,
        out_shape=jax.ShapeDtypeStruct((96, 3, 16, 16), jnp.float32),
    )(images)
    return jnp.concatenate([tok0, tok12], axis=0)


# speedup vs baseline: 1.1645x; 1.0214x over previous
import functools

import jax
import jax.numpy as jnp
from jax import lax
from jax.experimental import pallas as pl
from jax.experimental.pallas import tpu as pltpu
from jax.experimental.pallas import tpu_sc as plsc

_RING_SLICES = ((0, 18), (22, 26), (30, 34), (38, 42), (46, 64))

_NW = 32


def _pool_kernel(img_ref, out_ref):
    r = lax.broadcasted_iota(jnp.int32, (512, 128), 0)
    c = lax.broadcasted_iota(jnp.int32, (512, 128), 1)
    s4 = (r // 4 == c).astype(jnp.float32)
    s2 = (r[:256] // 2 == c[:256]).astype(jnp.float32)

    s4b = s4.astype(jnp.bfloat16)
    s2b = s2.astype(jnp.bfloat16)

    def split3(x):
        hi = x.astype(jnp.bfloat16)
        r = x - hi.astype(jnp.float32)
        mid = r.astype(jnp.bfloat16)
        lo = (r - mid.astype(jnp.float32)).astype(jnp.bfloat16)
        return hi, mid, lo

    def poolrows(s, parts):
        prods = [lax.dot_general(s, p, ((((0,), (0,))), ((), ())),
                                 preferred_element_type=jnp.float32)
                 for p in parts]
        return (prods[0] + prods[1]) + prods[2]

    def poolcols(x, s):
        parts = [lax.dot_general(p, s, ((((1,), (0,))), ((), ())),
                                 preferred_element_type=jnp.float32)
                 for p in split3(x)]
        return (parts[0] + parts[1]) + parts[2]

    for ch in range(3):
        img_parts = split3(img_ref[ch])
        sub_parts = [p[128:384, 128:384] for p in img_parts]
        rows2 = poolrows(s2b, sub_parts)
        p1 = jnp.floor(poolcols(rows2, s2b) * 0.25)
        rows4 = poolrows(s4b, img_parts)
        p2 = jnp.floor(poolcols(rows4, s4b) * 0.0625)

        ring_pos = {}
        for p, k in enumerate(k for a, b in _RING_SLICES for k in range(a, b)):
            ring_pos[k] = p
        for base, plane in ((0, p1), (48, p2)):
            for x in range(8):
                tcol = plane[:, 16 * x:16 * x + 16].reshape(8, 16, 16)
                for y in range(8):
                    k = y * 8 + x
                    if k in ring_pos:
                        out_ref[base + ring_pos[k], ch] = tcol[y]


_TOK0_PER_W = 64 // _NW


@functools.cache
def _make_l0_gather():
    @functools.partial(
        pl.kernel,
        out_type=jax.ShapeDtypeStruct((64, 3, 16, 16), jnp.float32),
        scratch_types=[
            pltpu.VMEM((1, 3, 16, 128), jnp.float32),
            pltpu.VMEM((_TOK0_PER_W, 3, 16, 16), jnp.float32),
            pltpu.SemaphoreType.DMA,
            pltpu.SemaphoreType.DMA,
        ],
        mesh=plsc.VectorSubcoreMesh(core_axis_name="c", subcore_axis_name="s"),
    )
    def _l0_gather(img_hbm, out_hbm, stage_v, tok_v, gsem, ssem):
        wid = lax.axis_index("s") * 2 + lax.axis_index("c")
        n0 = 2 * wid
        y, x0 = n0 // 8, n0 % 8
        win = jnp.where(x0 < 4, 128, 256)
        pltpu.async_copy(
            img_hbm.at[:, pl.ds(pl.multiple_of(192 + 16 * y, 16), 16),
                       pl.ds(pl.multiple_of(win, 128), 128)],
            stage_v.at[0], gsem).wait()
        stores = []
        for t in range(_TOK0_PER_W):
            col = pl.multiple_of(192 + 16 * (x0 + t) - win, 16)
            for c in range(3):
                for a in range(16):
                    v = stage_v[0, c, a, pl.ds(col, 16)]
                    tok_v[t, c, a, :] = v.astype(jnp.int32).astype(jnp.float32)
            stores.append(pltpu.async_copy(tok_v.at[t], out_hbm.at[n0 + t], ssem))
        for cp in stores:
            cp.wait()

    return _l0_gather


def kernel(images):
    tok0 = _make_l0_gather()(images)
    tok12 = pl.pallas_call(
        _pool_kernel,
        out_shape=jax.ShapeDtypeStruct((96, 3, 16, 16), jnp.float32),
    )(images)
    return jnp.concatenate([tok0, tok12], axis=0)
